# Initial kernel scaffold; baseline (speedup 1.0000x reference)
import jax
import jax.numpy as jnp
from jax.experimental import pallas as pl


def kernel(features_omics1, features_omics2, adj_spatial_omics1, adj_feature_omics1, adj_augmented_omics1, adj_spatial_omics2, adj_feature_omics2, adj_augmented_omics2, W_enc1_sp, W_enc1_ft, W_enc1_aug, W_enc2_sp, W_enc2_ft, W_enc2_aug, W_dec1, W_dec2, att1_w, att1_u, att2_w, att2_u, attc_w, attc_u):
    raise NotImplementedError("write your pallas kernel here")


# speedup vs baseline: 1.1877x; 1.1877x over previous
import functools

import jax
import jax.numpy as jnp
from jax.experimental import pallas as pl
from jax.experimental.pallas import tpu as pltpu

F32 = jnp.float32


def _mm_body(x_ref, y_ref, o_ref):
    @pl.when(pl.program_id(2) == 0)
    def _init():
        o_ref[...] = jnp.zeros_like(o_ref)

    o_ref[...] += jnp.dot(x_ref[...], y_ref[...], preferred_element_type=F32)


def _mm(a, b, bm=None, bk=None, bn=None):
    m, k = a.shape
    k2, n = b.shape
    assert k == k2, (a.shape, b.shape)
    bm = min(512, m) if bm is None else bm
    bk = min(1024, k) if bk is None else bk
    bn = min(512, n) if bn is None else bn
    assert m % bm == 0 and k % bk == 0 and n % bn == 0, (a.shape, b.shape, bm, bk, bn)
    return pl.pallas_call(
        _mm_body,
        grid=(m // bm, n // bn, k // bk),
        in_specs=[
            pl.BlockSpec((bm, bk), lambda i, j, kk: (i, kk)),
            pl.BlockSpec((bk, bn), lambda i, j, kk: (kk, j)),
        ],
        out_specs=pl.BlockSpec((bm, bn), lambda i, j, kk: (i, j)),
        out_shape=jax.ShapeDtypeStruct((m, n), F32),
        compiler_params=pltpu.CompilerParams(
            dimension_semantics=("parallel", "parallel", "arbitrary")),
    )(a, b)


def _att_body(e1s_ref, e1f_ref, e1a_ref, e2s_ref, e2f_ref, e2a_ref,
              w1_ref, u1_ref, w2_ref, u2_ref, wc_ref, uc_ref,
              l1_ref, l2_ref, co_ref, a1_ref, a2_ref, ac_ref):
    def score(e, w, u_t):
        v = jnp.tanh(jnp.dot(e, w, preferred_element_type=F32))
        return jnp.sum(v * u_t, axis=1, keepdims=True)

    def att3(es, ef, ea, w, u_t):
        ss, sf, sa = score(es, w, u_t), score(ef, w, u_t), score(ea, w, u_t)
        mx = jnp.maximum(jnp.maximum(ss, sf), sa)
        xs, xf, xa = jnp.exp(ss - mx), jnp.exp(sf - mx), jnp.exp(sa - mx)
        den = xs + xf + xa
        als, alf, ala = xs / den, xf / den, xa / den
        l = als * es + alf * ef + ala * ea
        return l, jnp.concatenate([als, alf, ala], axis=1)

    l1, a1 = att3(e1s_ref[...], e1f_ref[...], e1a_ref[...], w1_ref[...], u1_ref[...])
    l2, a2 = att3(e2s_ref[...], e2f_ref[...], e2a_ref[...], w2_ref[...], u2_ref[...])
    s1 = score(l1, wc_ref[...], uc_ref[...])
    s2 = score(l2, wc_ref[...], uc_ref[...])
    mx = jnp.maximum(s1, s2)
    x1, x2 = jnp.exp(s1 - mx), jnp.exp(s2 - mx)
    den = x1 + x2
    b1, b2 = x1 / den, x2 / den
    l1_ref[...] = l1
    l2_ref[...] = l2
    co_ref[...] = b1 * l1 + b2 * l2
    a1_ref[...] = a1
    a2_ref[...] = a2
    ac_ref[...] = jnp.concatenate([b1, b2], axis=1)


def _attention(e1s, e1f, e1a, e2s, e2f, e2a, w1, u1, w2, u2, wc, uc):
    n, h = e1s.shape
    bm = min(512, n)
    row = lambda i: (i, 0)
    fixed = lambda i: (0, 0)
    eb = pl.BlockSpec((bm, h), row)
    wb = pl.BlockSpec((h, h), fixed)
    ub = pl.BlockSpec((1, h), fixed)
    return pl.pallas_call(
        _att_body,
        grid=(n // bm,),
        in_specs=[eb, eb, eb, eb, eb, eb, wb, ub, wb, ub, wb, ub],
        out_specs=[
            pl.BlockSpec((bm, h), row),
            pl.BlockSpec((bm, h), row),
            pl.BlockSpec((bm, h), row),
            pl.BlockSpec((bm, 3), row),
            pl.BlockSpec((bm, 3), row),
            pl.BlockSpec((bm, 2), row),
        ],
        out_shape=[
            jax.ShapeDtypeStruct((n, h), F32),
            jax.ShapeDtypeStruct((n, h), F32),
            jax.ShapeDtypeStruct((n, h), F32),
            jax.ShapeDtypeStruct((n, 3), F32),
            jax.ShapeDtypeStruct((n, 3), F32),
            jax.ShapeDtypeStruct((n, 2), F32),
        ],
        compiler_params=pltpu.CompilerParams(
            dimension_semantics=("parallel",)),
    )(e1s, e1f, e1a, e2s, e2f, e2a, w1, u1.T, w2, u2.T, wc, uc.T)


def _pad_cols(x, mult=128):
    d = x.shape[1]
    p = (-d) % mult
    return jnp.pad(x, ((0, 0), (0, p))) if p else x


def _pad_rows(x, mult=128):
    d = x.shape[0]
    p = (-d) % mult
    return jnp.pad(x, ((0, p), (0, 0))) if p else x


def kernel(features_omics1, features_omics2, adj_spatial_omics1,
           adj_feature_omics1, adj_augmented_omics1, adj_spatial_omics2,
           adj_feature_omics2, adj_augmented_omics2, W_enc1_sp, W_enc1_ft,
           W_enc1_aug, W_enc2_sp, W_enc2_ft, W_enc2_aug, W_dec1, W_dec2,
           att1_w, att1_u, att2_w, att2_u, attc_w, attc_u):
    h = W_enc1_sp.shape[1]
    d1 = W_dec1.shape[1]

    a1s, a1f, a1a = adj_spatial_omics1, adj_feature_omics1, adj_augmented_omics1
    a2s, a2f, a2a = adj_spatial_omics2, adj_feature_omics2, adj_augmented_omics2

    x1p = _pad_cols(features_omics1)
    w1c = _pad_rows(jnp.concatenate([W_enc1_sp, W_enc1_ft, W_enc1_aug], axis=1))
    w2c = jnp.concatenate([W_enc2_sp, W_enc2_ft, W_enc2_aug], axis=1)
    y1 = _mm(x1p, w1c)
    y2 = _mm(features_omics2, w2c)

    e1s = _mm(a1s, y1[:, 0 * h:1 * h])
    e1f = _mm(a1f, y1[:, 1 * h:2 * h])
    e1a = _mm(a1a, y1[:, 2 * h:3 * h])
    e2s = _mm(a2s, y2[:, 0 * h:1 * h])
    e2f = _mm(a2f, y2[:, 1 * h:2 * h])
    e2a = _mm(a2a, y2[:, 2 * h:3 * h])

    l1, l2, comb, al1, al2, alc = _attention(
        e1s, e1f, e1a, e2s, e2f, e2a,
        att1_w, att1_u, att2_w, att2_u, attc_w, attc_u)

    wd1p = _pad_cols(W_dec1)
    md1 = _mm(wd1p, _pad_rows(W_enc1_sp), bm=h)
    md2 = _mm(W_dec2, W_enc2_sp, bm=h)

    u1 = _mm(a1s, jnp.concatenate([comb, l2], axis=1))
    u2 = _mm(a2s, jnp.concatenate([comb, l1], axis=1))

    rec1 = _mm(u1[:, :h], wd1p)[:, :d1]
    rec2 = _mm(u2[:, :h], W_dec2)
    x2r = _mm(a1s, _mm(u1[:, h:], md1))
    x1r = _mm(a2s, _mm(u2[:, h:], md2))

    return (l1, l2, comb, rec1, rec2, x1r, x2r, al1, al2, alc,
            e1s, e1f, e1a, e2s, e2f, e2a)
